# Initial kernel scaffold; baseline (speedup 1.0000x reference)
import jax
import jax.numpy as jnp
from jax.experimental import pallas as pl


def kernel(outputs, label_inds):
    raise NotImplementedError("write your pallas kernel here")


# speedup vs baseline: 19.1394x; 19.1394x over previous
import functools

import jax
import jax.numpy as jnp
from jax.experimental import pallas as pl
from jax.experimental.pallas import tpu as pltpu

_N = 4096
_D = 32
_K = 3
_BLK = 256


def _lmnn_kernel(x_blk_ref, x_full_ref, lab_blk_ref, lab_full_ref,
                 out_ref, pull_acc, push_acc, imp_acc):
    i = pl.program_id(0)
    nblk = pl.num_programs(0)

    @pl.when(i == 0)
    def _init():
        pull_acc[0] = 0.0
        push_acc[0] = 0.0
        imp_acc[0] = 0

    x_blk = x_blk_ref[...]
    x_full = x_full_ref[...]
    lab_blk = lab_blk_ref[...]
    lab_full = lab_full_ref[...]

    g = jax.lax.dot_general(
        x_blk, x_full, (((1,), (1,)), ((), ())),
        preferred_element_type=jnp.float32)
    a2_blk = jnp.sum(x_blk * x_blk, axis=1, keepdims=True)
    a2_full = jnp.sum(x_full * x_full, axis=1)[None, :]
    d = jnp.maximum(a2_blk + a2_full - 2.0 * g, 0.0)

    col = jax.lax.broadcasted_iota(jnp.int32, (_BLK, _N), 1)
    row_g = jax.lax.broadcasted_iota(jnp.int32, (_BLK, _N), 0) + i * _BLK
    same = lab_blk == lab_full
    valid = same & (col != row_g)
    inf = jnp.float32(jnp.inf)
    dd = jnp.where(valid, d, inf)

    row_id = row_g[:, :1]

    pull = jnp.zeros((_BLK, 1), jnp.float32)
    tmax = jnp.full((_BLK, 1), -inf)
    targ_vals = []
    targ_isdiag = []
    for _ in range(_K):
        m = jnp.min(dd, axis=1, keepdims=True)
        idx = jnp.min(jnp.where(dd == m, col, _N), axis=1, keepdims=True)
        hit = col == idx
        tval = jnp.sum(jnp.where(hit, d, 0.0), axis=1, keepdims=True)
        dd = jnp.where(hit, inf, dd)
        pull = pull + tval
        tmax = jnp.maximum(tmax, tval)
        targ_vals.append(tval)
        targ_isdiag.append(idx == row_id)

    margin = 1.0 + tmax

    push_full = jnp.sum(jnp.maximum(margin - d, 0.0), axis=1, keepdims=True)
    corr = margin
    for tval, isdiag in zip(targ_vals, targ_isdiag):
        corr = corr + jnp.where(isdiag, 0.0, jnp.maximum(margin - tval, 0.0))
    push = push_full - corr

    imp = jnp.any(jnp.logical_not(same) & (d < margin))

    pull_acc[0] = pull_acc[0] + jnp.sum(pull)
    push_acc[0] = push_acc[0] + jnp.sum(push)
    imp_acc[0] = imp_acc[0] | imp.astype(jnp.int32)

    @pl.when(i == nblk - 1)
    def _fin():
        p = pull_acc[0]
        s = push_acc[0]
        total = jnp.where(imp_acc[0] > 0, (p + s) / _N, p / _N)
        out_ref[...] = jnp.broadcast_to(total, (1, 1))


def kernel(outputs, label_inds):
    lab_col = label_inds.reshape(_N, 1)
    lab_row = label_inds.reshape(1, _N)
    grid = _N // _BLK
    out = pl.pallas_call(
        _lmnn_kernel,
        grid=(grid,),
        in_specs=[
            pl.BlockSpec((_BLK, _D), lambda i: (i, 0)),
            pl.BlockSpec((_N, _D), lambda i: (0, 0)),
            pl.BlockSpec((_BLK, 1), lambda i: (i, 0)),
            pl.BlockSpec((1, _N), lambda i: (0, 0)),
        ],
        out_specs=pl.BlockSpec((1, 1), lambda i: (0, 0)),
        out_shape=jax.ShapeDtypeStruct((1, 1), jnp.float32),
        scratch_shapes=[
            pltpu.SMEM((1,), jnp.float32),
            pltpu.SMEM((1,), jnp.float32),
            pltpu.SMEM((1,), jnp.int32),
        ],
        compiler_params=pltpu.CompilerParams(
            dimension_semantics=("arbitrary",)),
    )(outputs, outputs, lab_col, lab_row)
    return out[0, 0]


# speedup vs baseline: 28.5412x; 1.4912x over previous
import jax
import jax.numpy as jnp
from jax.experimental import pallas as pl
from jax.experimental.pallas import tpu as pltpu

_N = 4096
_D = 32
_K = 3
_BLK = 256


def _lmnn_kernel(x_blk_ref, x_full_ref, lab_blk_ref, lab_full_ref,
                 out_ref, pull_acc, push_acc, imp_acc,
                 margin_ref, pullrow_ref, corrrow_ref):
    i = pl.program_id(0)
    nblk = pl.num_programs(0)

    @pl.when(i == 0)
    def _init():
        pull_acc[0] = 0.0
        push_acc[0] = 0.0
        imp_acc[0] = 0

    x_blk = x_blk_ref[...]
    x_full = x_full_ref[...]
    lab_blk = lab_blk_ref[...]
    lab_full = lab_full_ref[...]
    inf = jnp.float32(jnp.inf)

    a2b = jnp.sum(x_blk * x_blk, axis=1, keepdims=True)
    a2f = jnp.sum(x_full * x_full, axis=1, keepdims=True)
    aug_b = jnp.concatenate(
        [x_blk * -2.0, a2b, jnp.ones((_BLK, 1), jnp.float32)], axis=1)
    aug_f = jnp.concatenate(
        [x_full, jnp.ones((_N, 1), jnp.float32), a2f], axis=1)
    g = jax.lax.dot_general(
        aug_b, aug_f, (((1,), (1,)), ((), ())),
        preferred_element_type=jnp.float32)
    d = jnp.maximum(g, 0.0)

    col = jax.lax.broadcasted_iota(jnp.int32, (_BLK, _N), 1)
    rowl = jax.lax.broadcasted_iota(jnp.int32, (_BLK, _N), 0)
    offd = (col - rowl) != i * _BLK
    same = lab_blk == lab_full
    dd = jnp.where(same & offd, d, inf)

    m1 = jnp.min(dd, axis=1, keepdims=True)
    e1 = dd == m1
    c1 = jnp.sum(jnp.where(e1, 1.0, 0.0), axis=1, keepdims=True)
    m2 = jnp.min(jnp.where(e1, inf, dd), axis=1, keepdims=True)
    le = dd <= m2
    c12 = jnp.sum(jnp.where(le, 1.0, 0.0), axis=1, keepdims=True)
    m3 = jnp.min(jnp.where(le, inf, dd), axis=1, keepdims=True)

    k2 = jnp.where(c1 >= 1.5, m1, m2)
    k3 = jnp.where(c1 >= 2.5, m1, jnp.where(c12 >= 2.5, m2, m3))

    margin_v = 1.0 + k3
    sum3 = m1 + k2 + k3
    margin_ref[...] = margin_v
    pullrow_ref[...] = sum3
    corrrow_ref[...] = 4.0 * margin_v - sum3

    @pl.when(jnp.any(k3 == inf))
    def _slow():
        row_g = rowl + i * _BLK
        row_id = row_g[:, :1]
        ddx = dd
        colx = col
        pull = jnp.zeros((_BLK, 1), jnp.float32)
        tmax = jnp.full((_BLK, 1), -inf)
        corr = jnp.zeros((_BLK, 1), jnp.float32)
        tvals = []
        isdiags = []
        for _ in range(_K):
            m = jnp.min(ddx, axis=1, keepdims=True)
            idx = jnp.min(jnp.where(ddx == m, colx, _N), axis=1,
                          keepdims=True)
            hit = col == idx
            tval = jnp.sum(jnp.where(hit, d, 0.0), axis=1, keepdims=True)
            ddx = jnp.where(hit, inf, ddx)
            colx = jnp.where(hit, _N, colx)
            pull = pull + tval
            tmax = jnp.maximum(tmax, tval)
            tvals.append(tval)
            isdiags.append(idx == row_id)
        margin_s = 1.0 + tmax
        corr = margin_s
        for tval, isdiag in zip(tvals, isdiags):
            corr = corr + jnp.where(
                isdiag, 0.0, jnp.maximum(margin_s - tval, 0.0))
        margin_ref[...] = margin_s
        pullrow_ref[...] = pull
        corrrow_ref[...] = corr

    margin = margin_ref[...]
    t = margin - d
    push_full = jnp.sum(jnp.maximum(t, 0.0), axis=1, keepdims=True)
    imp = jnp.max(jnp.where(same, -inf, t)) > 0.0

    pull_acc[0] = pull_acc[0] + jnp.sum(pullrow_ref[...])
    push_acc[0] = push_acc[0] + jnp.sum(push_full - corrrow_ref[...])
    imp_acc[0] = imp_acc[0] | imp.astype(jnp.int32)

    @pl.when(i == nblk - 1)
    def _fin():
        p = pull_acc[0]
        s = push_acc[0]
        total = jnp.where(imp_acc[0] > 0, (p + s) / _N, p / _N)
        out_ref[...] = jnp.broadcast_to(total, (1, 1))


def kernel(outputs, label_inds):
    lab_col = label_inds.reshape(_N, 1)
    lab_row = label_inds.reshape(1, _N)
    grid = _N // _BLK
    out = pl.pallas_call(
        _lmnn_kernel,
        grid=(grid,),
        in_specs=[
            pl.BlockSpec((_BLK, _D), lambda i: (i, 0)),
            pl.BlockSpec((_N, _D), lambda i: (0, 0)),
            pl.BlockSpec((_BLK, 1), lambda i: (i, 0)),
            pl.BlockSpec((1, _N), lambda i: (0, 0)),
        ],
        out_specs=pl.BlockSpec((1, 1), lambda i: (0, 0)),
        out_shape=jax.ShapeDtypeStruct((1, 1), jnp.float32),
        scratch_shapes=[
            pltpu.SMEM((1,), jnp.float32),
            pltpu.SMEM((1,), jnp.float32),
            pltpu.SMEM((1,), jnp.int32),
            pltpu.VMEM((_BLK, 1), jnp.float32),
            pltpu.VMEM((_BLK, 1), jnp.float32),
            pltpu.VMEM((_BLK, 1), jnp.float32),
        ],
        compiler_params=pltpu.CompilerParams(
            dimension_semantics=("arbitrary",)),
    )(outputs, outputs, lab_col, lab_row)
    return out[0, 0]
